# Initial kernel scaffold; baseline (speedup 1.0000x reference)
import jax
import jax.numpy as jnp
from jax.experimental import pallas as pl


def kernel(inputs, W):
    raise NotImplementedError("write your pallas kernel here")


# speedup vs baseline: 1.6988x; 1.6988x over previous
import jax
import jax.numpy as jnp
from jax.experimental import pallas as pl
from jax.experimental.pallas import tpu as pltpu

_NE = 1024
_D = 64
_B = 16
_L = 1024
_ROWS = _B * _L
_BLK = 1024
_GRID = _ROWS // _BLK


def _vq_body(xt_ref, w_ref, wt_ref, enc_ref, q_ref, loss_ref, perp_ref,
             counts_scr, sq_scr, b_scr):
    i = pl.program_id(0)

    @pl.when(i == 0)
    def _init():
        counts_scr[...] = jnp.zeros_like(counts_scr)
        sq_scr[0, 0] = 0.0
        b_scr[...] = jnp.sum(w_ref[...] * w_ref[...], axis=1)[None, :]

    xt = xt_ref[0]
    w = w_ref[...]
    wt = wt_ref[...]

    m = jax.lax.dot_general(xt, wt, (((0,), (0,)), ((), ())),
                            preferred_element_type=jnp.float32)
    a = jnp.sum(xt * xt, axis=0)[:, None]
    b = b_scr[0]
    d = a + b[None, :] - 2.0 * m

    dmin = jnp.min(d, axis=1)
    iota = jax.lax.broadcasted_iota(jnp.int32, (1, _NE), 1).astype(jnp.float32)
    idx = jnp.min(jnp.where(d == dmin[:, None], iota, float(_NE)), axis=1)
    enc = (iota == idx[:, None]).astype(jnp.float32)

    enc_ref[...] = enc
    q_ref[0] = jax.lax.dot_general(w, enc, (((0,), (1,)), ((), ())),
                                   preferred_element_type=jnp.float32)

    counts_scr[...] += jnp.sum(enc, axis=0)[None, :]
    sq_scr[0, 0] += jnp.sum(dmin)

    @pl.when(i == _GRID - 1)
    def _fin():
        n_elems = float(_ROWS * _D)
        loss_ref[0, 0] = 1.25 * sq_scr[0, 0] / n_elems
        p = counts_scr[...] / float(_ROWS)
        ent = jnp.sum(p * jnp.log(p + 1e-10))
        perp_ref[0, 0] = jnp.exp(-ent)


def kernel(inputs, W):
    wt = W.T
    enc, q, loss, perp = pl.pallas_call(
        _vq_body,
        grid=(_GRID,),
        in_specs=[
            pl.BlockSpec((1, _D, _BLK), lambda i: (i, 0, 0)),
            pl.BlockSpec((_NE, _D), lambda i: (0, 0)),
            pl.BlockSpec((_D, _NE), lambda i: (0, 0)),
        ],
        out_specs=[
            pl.BlockSpec((_BLK, _NE), lambda i: (i, 0)),
            pl.BlockSpec((1, _D, _BLK), lambda i: (i, 0, 0)),
            pl.BlockSpec(memory_space=pltpu.SMEM),
            pl.BlockSpec(memory_space=pltpu.SMEM),
        ],
        out_shape=[
            jax.ShapeDtypeStruct((_ROWS, _NE), jnp.float32),
            jax.ShapeDtypeStruct((_B, _D, _L), jnp.float32),
            jax.ShapeDtypeStruct((1, 1), jnp.float32),
            jax.ShapeDtypeStruct((1, 1), jnp.float32),
        ],
        scratch_shapes=[
            pltpu.VMEM((1, _NE), jnp.float32),
            pltpu.SMEM((1, 1), jnp.float32),
            pltpu.VMEM((1, _NE), jnp.float32),
        ],
    )(inputs, W, wt)
    return (loss[0, 0], q, perp[0, 0], enc)
